# Initial kernel scaffold; baseline (speedup 1.0000x reference)
import jax
import jax.numpy as jnp
from jax.experimental import pallas as pl


def kernel(input_feat, segment_ids, W_in, b_in, W_down, b_down, W_skip, b_skip):
    raise NotImplementedError("write your pallas kernel here")


# speedup vs baseline: 2.0028x; 2.0028x over previous
import functools

import jax
import jax.numpy as jnp
from jax import lax
from jax.experimental import pallas as pl
from jax.experimental.pallas import tpu as pltpu
from jax.experimental.pallas import tpu_sc as plsc

N = 160000
D = 128
S = 80000

NC = 2
NS = 16
NW = NC * NS
NPW = N // NW
CHUNK = 200
NCHUNK = NPW // CHUNK
CAP = 128
EXT = 16
SP = S + 16
G = 128
NG = NPW // G
GT = NPW - NG * G

_mesh = plsc.VectorSubcoreMesh(core_axis_name="c", subcore_axis_name="s")


def _wid():
    return lax.axis_index("s") * NC + lax.axis_index("c")


@functools.partial(
    pl.kernel,
    out_type=(jax.ShapeDtypeStruct((SP, D), jnp.float32),
              jax.ShapeDtypeStruct((2 * NW, D), jnp.float32),
              jax.ShapeDtypeStruct((2 * NW, 16), jnp.float32)),
    mesh=_mesh,
    scratch_types=[
        pltpu.VMEM((CHUNK, D), jnp.float32),
        pltpu.VMEM((CHUNK + 16,), jnp.int32),
        pltpu.VMEM((CAP, D), jnp.float32),
        pltpu.VMEM((CAP,), jnp.int32),
        pltpu.VMEM((16,), jnp.int32),
        pltpu.VMEM((2, D), jnp.float32),
        pltpu.VMEM((2, 16), jnp.float32),
        pltpu.SemaphoreType.DMA,
    ],
)
def _seg_scan_k(x_hbm, ids_hbm, means_hbm, side_hbm, meta_hbm,
                xbuf, idbuf, runbuf, ridbuf, firstbuf, sidebuf,
                metabuf, sem):
    wid = _wid()
    base = wid * NPW

    pltpu.sync_copy(ids_hbm.at[pl.ds(pl.multiple_of(base, 8), 16)], firstbuf)
    first_id = firstbuf[...][0]

    lane_iota = lax.iota(jnp.int32, 16)

    def scatter_flush():
        pltpu.async_copy(runbuf, means_hbm.at[ridbuf], sem).wait()

    def store_state(slot, rid, cnt, acc, pend):
        pend = jnp.where(lane_iota == jnp.bitwise_and(slot, 15), rid, pend)
        inv = (jnp.ones((16,), jnp.float32)
               / jnp.full((16,), jnp.maximum(cnt, jnp.float32(1.0)),
                          jnp.float32))
        for j in range(8):
            runbuf[slot, pl.ds(16 * j, 16)] = acc[j] * inv
        blk16 = pl.multiple_of(jnp.right_shift(slot, 4) * 16, 16)
        ridbuf[pl.ds(blk16, 16)] = pend
        return pend

    def row_step(rid, xs, st):
        (cur_id, cnt, own, slot, pend, fact, fcnt) = st[:7]
        acc = st[7:15]
        facc = st[15:]
        same = rid == cur_id
        do_flush = jnp.logical_and(jnp.logical_not(same), own)
        slot = slot + jnp.where(do_flush, 1, 0)

        @pl.when(slot == CAP)
        def _():
            scatter_flush()

        slot = jnp.where(slot == CAP, 0, slot)
        nacc = tuple(jnp.where(same, a + x, x) for a, x in zip(acc, xs))
        ncnt = jnp.where(same, cnt + jnp.float32(1.0), jnp.float32(1.0))
        nown = jnp.logical_or(own, jnp.logical_not(same))
        pend = store_state(slot, rid, ncnt, nacc, pend)
        nfact = jnp.logical_and(fact, rid == first_id)
        nfacc = tuple(f + jnp.where(nfact, x, jnp.float32(0.0))
                      for f, x in zip(facc, xs))
        nfcnt = fcnt + jnp.where(nfact, jnp.float32(1.0), jnp.float32(0.0))
        return (rid, ncnt, nown, slot, pend, nfact, nfcnt) + nacc + nfacc

    def chunk_body(ch, st):
        off = pl.multiple_of(base + ch * CHUNK, 8)
        pltpu.sync_copy(x_hbm.at[pl.ds(off, CHUNK), :], xbuf)
        pltpu.sync_copy(ids_hbm.at[pl.ds(off, CHUNK)], idbuf.at[pl.ds(0, CHUNK)])

        def row_body(r, st):
            rid = idbuf[pl.ds(r, 16)][0]
            xs = tuple(xbuf[r, pl.ds(16 * j, 16)] for j in range(8))
            return row_step(rid, xs, st)

        return lax.fori_loop(0, CHUNK, row_body, st)

    zero8 = tuple(jnp.zeros((16,), jnp.float32) for _ in range(8))
    st0 = ((jnp.int32(-1), jnp.float32(0.0), jnp.bool_(False), jnp.int32(0),
            jnp.zeros((16,), jnp.int32), jnp.bool_(True), jnp.float32(0.0))
           + zero8 + zero8)
    st = lax.fori_loop(0, NCHUNK, chunk_body, st0)
    (cur_id, cnt, own, slot) = st[0], st[1], st[2], st[3]
    acc = st[7:15]
    fcnt = st[6]
    facc = st[15:]

    for j in range(8):
        sidebuf[0, pl.ds(16 * j, 16)] = facc[j]
        sidebuf[1, pl.ds(16 * j, 16)] = acc[j]
    fidf = jnp.float32(first_id)
    cidf = jnp.float32(cur_id)
    ownf = jnp.where(own, jnp.float32(1.0), jnp.float32(0.0))
    meta0 = jnp.where(lane_iota == 0, fcnt,
                      jnp.where(lane_iota == 1, fidf,
                                jnp.where(lane_iota == 2, jnp.float32(1.0),
                                          jnp.float32(0.0))))
    meta1 = jnp.where(lane_iota == 0, cnt,
                      jnp.where(lane_iota == 1, cidf,
                                jnp.where(lane_iota == 2, ownf,
                                          jnp.float32(0.0))))
    metabuf[0, pl.ds(0, 16)] = meta0
    metabuf[1, pl.ds(0, 16)] = meta1
    pltpu.sync_copy(sidebuf, side_hbm.at[pl.ds(2 * wid, 2), :])
    pltpu.sync_copy(metabuf, meta_hbm.at[pl.ds(2 * wid, 2), :])

    for kb in range(CAP // 16):
        idxv = lane_iota + kb * 16
        old = ridbuf[pl.ds(kb * 16, 16)]
        ridbuf[pl.ds(kb * 16, 16)] = jnp.where(idxv >= slot, jnp.int32(S), old)
    scatter_flush()


@functools.partial(
    pl.kernel,
    out_type=(jax.ShapeDtypeStruct((2 * NW, D), jnp.float32),
              jax.ShapeDtypeStruct((1, 2 * NW), jnp.float32)),
    mesh=_mesh,
    scratch_types=[
        pltpu.VMEM((2 * NW, D), jnp.float32),
        pltpu.VMEM((2 * NW, 16), jnp.float32),
        pltpu.VMEM((2 * NW, D), jnp.float32),
        pltpu.VMEM((1, 2 * NW), jnp.float32),
    ],
)
def _merge_k(side_hbm, meta_hbm, bmean_hbm, bid_hbm, sbuf, mbuf, obuf, oidbuf):
    pltpu.sync_copy(side_hbm, sbuf)
    pltpu.sync_copy(meta_hbm, mbuf)
    lane_iota = lax.iota(jnp.int32, 16)

    def zero_row(e, _):
        for j in range(8):
            obuf[e, pl.ds(16 * j, 16)] = jnp.zeros((16,), jnp.float32)
        return 0

    lax.fori_loop(0, 2 * NW, zero_row, 0)

    def store_state(slot, idf, cnt, acc, pend):
        pend = jnp.where(lane_iota == jnp.bitwise_and(slot, 15), idf, pend)
        inv = (jnp.ones((16,), jnp.float32)
               / jnp.full((16,), jnp.maximum(cnt, jnp.float32(1.0)),
                          jnp.float32))
        for j in range(8):
            obuf[slot, pl.ds(16 * j, 16)] = acc[j] * inv
        blk16 = pl.multiple_of(jnp.right_shift(slot, 4) * 16, 16)
        oidbuf[0, pl.ds(blk16, 16)] = pend
        return pend

    def ent_step(e, st):
        (cur_id, cnt, own, slot, pend) = st[:5]
        acc = st[5:]
        mrow = mbuf[e, pl.ds(0, 16)]
        ecnt, eid, evalid = mrow[0], mrow[1], mrow[2]
        valid = evalid > jnp.float32(0.5)
        xs = tuple(sbuf[e, pl.ds(16 * j, 16)] for j in range(8))
        same = jnp.logical_or(eid == cur_id, jnp.logical_not(valid))
        do_flush = jnp.logical_and(jnp.logical_not(same), own)
        slot = slot + jnp.where(do_flush, 1, 0)
        gate = jnp.logical_and(valid, jnp.bool_(True))
        add = jnp.where(gate, jnp.float32(1.0), jnp.float32(0.0))
        nacc = tuple(jnp.where(same, a + x * add, x)
                     for a, x in zip(acc, xs))
        ncnt = jnp.where(same, cnt + ecnt * add, ecnt)
        nid = jnp.where(valid, eid, cur_id)
        nown = jnp.logical_or(own, jnp.logical_not(same))
        pend = store_state(slot, nid, ncnt, nacc, pend)
        return (nid, ncnt, nown, slot, pend) + nacc

    zero8 = tuple(jnp.zeros((16,), jnp.float32) for _ in range(8))
    st0 = ((jnp.float32(-1.0), jnp.float32(0.0), jnp.bool_(False),
            jnp.int32(0), jnp.zeros((16,), jnp.float32)) + zero8)
    st = lax.fori_loop(0, 2 * NW, ent_step, st0)
    own, slot, pend = st[2], st[3], st[4]
    slot_eff = slot + jnp.where(own, 1, 0)

    for kb in range((2 * NW) // 16):
        idxv = lane_iota + kb * 16
        old = oidbuf[0, pl.ds(kb * 16, 16)]
        oidbuf[0, pl.ds(kb * 16, 16)] = jnp.where(idxv >= slot_eff,
                                                  jnp.float32(-1.0), old)
    pltpu.sync_copy(obuf, bmean_hbm)
    pltpu.sync_copy(oidbuf, bid_hbm)


def _proj_body(x_ref, w1_ref, w2_ref, b1_ref, b2_ref, o_ref):
    hi = jax.lax.Precision.HIGHEST
    w = jnp.dot(w1_ref[...], w2_ref[...], precision=hi,
                preferred_element_type=jnp.float32)
    b = jnp.dot(b1_ref[...], w2_ref[...], precision=hi,
                preferred_element_type=jnp.float32) + b2_ref[...]
    o_ref[...] = jnp.dot(x_ref[...], w, precision=hi,
                         preferred_element_type=jnp.float32) + b


def _proj(x, w1, w2, b1, b2, blk):
    rows = x.shape[0]
    grid = rows // blk
    return pl.pallas_call(
        _proj_body,
        grid=(grid,),
        in_specs=[
            pl.BlockSpec((blk, D), lambda i: (i, 0)),
            pl.BlockSpec((D, D), lambda i: (0, 0)),
            pl.BlockSpec((D, D), lambda i: (0, 0)),
            pl.BlockSpec((1, D), lambda i: (0, 0)),
            pl.BlockSpec((1, D), lambda i: (0, 0)),
        ],
        out_specs=pl.BlockSpec((blk, D), lambda i: (i, 0)),
        out_shape=jax.ShapeDtypeStruct((rows, D), jnp.float32),
    )(x, w1, w2, b1.reshape(1, D), b2.reshape(1, D))


def _proj_patch_body(x_ref, bm_ref, bi_ref, w1_ref, w2_ref, b1_ref, b2_ref,
                     o_ref):
    hi = jax.lax.Precision.HIGHEST
    i = pl.program_id(0)
    blk = x_ref.shape[0]
    w = jnp.dot(w1_ref[...], w2_ref[...], precision=hi,
                preferred_element_type=jnp.float32)
    b = jnp.dot(b1_ref[...], w2_ref[...], precision=hi,
                preferred_element_type=jnp.float32) + b2_ref[...]
    rowid = (jax.lax.broadcasted_iota(jnp.int32, (blk, 2 * NW), 0)
             + blk * i).astype(jnp.float32)
    eq = (rowid == bi_ref[...]).astype(jnp.float32)
    corr = jnp.dot(eq, bm_ref[...], precision=hi,
                   preferred_element_type=jnp.float32)
    hasb = jnp.sum(eq, axis=1, keepdims=True)
    means = jnp.where(hasb > 0, corr, x_ref[...])
    o_ref[...] = jnp.dot(means, w, precision=hi,
                         preferred_element_type=jnp.float32) + b


def _proj_patch(x, bm, bi, w1, w2, b1, b2, blk):
    rows = x.shape[0]
    grid = rows // blk
    return pl.pallas_call(
        _proj_patch_body,
        grid=(grid,),
        in_specs=[
            pl.BlockSpec((blk, D), lambda i: (i, 0)),
            pl.BlockSpec((2 * NW, D), lambda i: (0, 0)),
            pl.BlockSpec((1, 2 * NW), lambda i: (0, 0)),
            pl.BlockSpec((D, D), lambda i: (0, 0)),
            pl.BlockSpec((D, D), lambda i: (0, 0)),
            pl.BlockSpec((1, D), lambda i: (0, 0)),
            pl.BlockSpec((1, D), lambda i: (0, 0)),
        ],
        out_specs=pl.BlockSpec((blk, D), lambda i: (i, 0)),
        out_shape=jax.ShapeDtypeStruct((rows, D), jnp.float32),
    )(x, bm, bi, w1, w2, b1.reshape(1, D), b2.reshape(1, D))


@functools.partial(
    pl.kernel,
    out_type=jax.ShapeDtypeStruct((N, D), jnp.float32),
    mesh=_mesh,
    scratch_types=[
        pltpu.VMEM((G,), jnp.int32),
        pltpu.VMEM((G, D), jnp.float32),
        pltpu.VMEM((G, D), jnp.float32),
        pltpu.SemaphoreType.DMA,
    ],
)
def _gather_add_k(part_hbm, pd_hbm, ids_hbm, out_hbm, idbuf, pbuf, gbuf, sem):
    wid = _wid()
    base = wid * NPW

    def do_chunk(pos, g):
        pos = pl.multiple_of(pos, 8)
        pltpu.sync_copy(ids_hbm.at[pl.ds(pos, g)], idbuf.at[pl.ds(0, g)])
        pltpu.sync_copy(part_hbm.at[pl.ds(pos, g), :], pbuf.at[pl.ds(0, g), :])
        pltpu.async_copy(pd_hbm.at[idbuf.at[pl.ds(0, g)]],
                         gbuf.at[pl.ds(0, g), :], sem).wait()

        def row_body(r, _):
            for j in range(8):
                sl = pl.ds(16 * j, 16)
                pbuf[r, sl] = pbuf[r, sl] + gbuf[r, sl]
            return 0

        lax.fori_loop(0, g, row_body, 0)
        pltpu.sync_copy(pbuf.at[pl.ds(0, g), :], out_hbm.at[pl.ds(pos, g), :])

    def chunk_body(ch, _):
        do_chunk(base + ch * G, G)
        return 0

    lax.fori_loop(0, NG, chunk_body, 0)
    do_chunk(base + NG * G, GT)


def kernel(input_feat, segment_ids, W_in, b_in, W_down, b_down, W_skip, b_skip):
    ids32 = segment_ids.astype(jnp.int32)
    means, sides, metas = _seg_scan_k(input_feat, ids32)
    bmeans, bids = _merge_k(sides, metas)
    partial = _proj(input_feat, W_in, W_skip, b_in, b_skip, 2000)
    pd = _proj_patch(means[:S], bmeans, bids,
                     W_in, W_down, b_in, b_down, 2000)
    return _gather_add_k(partial, pd, ids32)


# speedup vs baseline: 2.1279x; 1.0625x over previous
import functools

import jax
import jax.numpy as jnp
from jax import lax
from jax.experimental import pallas as pl
from jax.experimental.pallas import tpu as pltpu
from jax.experimental.pallas import tpu_sc as plsc

N = 160000
D = 128
S = 80000

NC = 2
NS = 16
NW = NC * NS
NPW = N // NW
CHUNK = 200
NCHUNK = NPW // CHUNK
CAP = 128
EXT = 16
SP = S + 16
G = 128
NG = NPW // G
GT = NPW - NG * G

_mesh = plsc.VectorSubcoreMesh(core_axis_name="c", subcore_axis_name="s")


def _wid():
    return lax.axis_index("s") * NC + lax.axis_index("c")


@functools.partial(
    pl.kernel,
    out_type=(jax.ShapeDtypeStruct((SP, D), jnp.float32),
              jax.ShapeDtypeStruct((2 * NW, D), jnp.float32),
              jax.ShapeDtypeStruct((2 * NW, 16), jnp.float32)),
    mesh=_mesh,
    scratch_types=[
        pltpu.VMEM((CHUNK, D), jnp.float32),
        pltpu.VMEM((CHUNK + 16,), jnp.int32),
        pltpu.VMEM((CAP, D), jnp.float32),
        pltpu.VMEM((CAP,), jnp.int32),
        pltpu.VMEM((16,), jnp.int32),
        pltpu.VMEM((2, D), jnp.float32),
        pltpu.VMEM((2, 16), jnp.float32),
        pltpu.SemaphoreType.DMA,
    ],
)
def _seg_scan_k(x_hbm, ids_hbm, means_hbm, side_hbm, meta_hbm,
                xbuf, idbuf, runbuf, ridbuf, firstbuf, sidebuf,
                metabuf, sem):
    wid = _wid()
    base = wid * NPW

    pltpu.sync_copy(ids_hbm.at[pl.ds(pl.multiple_of(base, 8), 16)], firstbuf)
    first_id = firstbuf[...][0]

    lane_iota = lax.iota(jnp.int32, 16)

    def scatter_flush():
        pltpu.async_copy(runbuf, means_hbm.at[ridbuf], sem).wait()

    def store_state(slot, rid, cnt, acc, pend):
        pend = jnp.where(lane_iota == jnp.bitwise_and(slot, 15), rid, pend)
        inv = (jnp.ones((16,), jnp.float32)
               / jnp.full((16,), jnp.maximum(cnt, jnp.float32(1.0)),
                          jnp.float32))
        for j in range(8):
            runbuf[slot, pl.ds(16 * j, 16)] = acc[j] * inv
        blk16 = pl.multiple_of(jnp.right_shift(slot, 4) * 16, 16)
        ridbuf[pl.ds(blk16, 16)] = pend
        return pend

    def row_step(rid, xs, st):
        (cur_id, cnt, own, slot, pend, fact, fcnt) = st[:7]
        acc = st[7:15]
        facc = st[15:]
        same = rid == cur_id
        do_flush = jnp.logical_and(jnp.logical_not(same), own)
        slot = slot + jnp.where(do_flush, 1, 0)

        @pl.when(slot == CAP)
        def _():
            scatter_flush()

        slot = jnp.where(slot == CAP, 0, slot)
        nacc = tuple(jnp.where(same, a + x, x) for a, x in zip(acc, xs))
        ncnt = jnp.where(same, cnt + jnp.float32(1.0), jnp.float32(1.0))
        nown = jnp.logical_or(own, jnp.logical_not(same))
        pend = store_state(slot, rid, ncnt, nacc, pend)
        nfact = jnp.logical_and(fact, rid == first_id)
        nfacc = tuple(f + jnp.where(nfact, x, jnp.float32(0.0))
                      for f, x in zip(facc, xs))
        nfcnt = fcnt + jnp.where(nfact, jnp.float32(1.0), jnp.float32(0.0))
        return (rid, ncnt, nown, slot, pend, nfact, nfcnt) + nacc + nfacc

    def chunk_body(ch, st):
        off = pl.multiple_of(base + ch * CHUNK, 8)
        pltpu.sync_copy(x_hbm.at[pl.ds(off, CHUNK), :], xbuf)
        pltpu.sync_copy(ids_hbm.at[pl.ds(off, CHUNK)], idbuf.at[pl.ds(0, CHUNK)])

        def row_body(r, st):
            rid = idbuf[pl.ds(r, 16)][0]
            xs = tuple(xbuf[r, pl.ds(16 * j, 16)] for j in range(8))
            return row_step(rid, xs, st)

        return lax.fori_loop(0, CHUNK, row_body, st)

    zero8 = tuple(jnp.zeros((16,), jnp.float32) for _ in range(8))
    st0 = ((jnp.int32(-1), jnp.float32(0.0), jnp.bool_(False), jnp.int32(0),
            jnp.zeros((16,), jnp.int32), jnp.bool_(True), jnp.float32(0.0))
           + zero8 + zero8)
    st = lax.fori_loop(0, NCHUNK, chunk_body, st0)
    (cur_id, cnt, own, slot) = st[0], st[1], st[2], st[3]
    acc = st[7:15]
    fcnt = st[6]
    facc = st[15:]

    for j in range(8):
        sidebuf[0, pl.ds(16 * j, 16)] = facc[j]
        sidebuf[1, pl.ds(16 * j, 16)] = acc[j]
    fidf = jnp.float32(first_id)
    cidf = jnp.float32(cur_id)
    ownf = jnp.where(own, jnp.float32(1.0), jnp.float32(0.0))
    meta0 = jnp.where(lane_iota == 0, fcnt,
                      jnp.where(lane_iota == 1, fidf,
                                jnp.where(lane_iota == 2, jnp.float32(1.0),
                                          jnp.float32(0.0))))
    meta1 = jnp.where(lane_iota == 0, cnt,
                      jnp.where(lane_iota == 1, cidf,
                                jnp.where(lane_iota == 2, ownf,
                                          jnp.float32(0.0))))
    metabuf[0, pl.ds(0, 16)] = meta0
    metabuf[1, pl.ds(0, 16)] = meta1
    pltpu.sync_copy(sidebuf, side_hbm.at[pl.ds(2 * wid, 2), :])
    pltpu.sync_copy(metabuf, meta_hbm.at[pl.ds(2 * wid, 2), :])

    for kb in range(CAP // 16):
        idxv = lane_iota + kb * 16
        old = ridbuf[pl.ds(kb * 16, 16)]
        ridbuf[pl.ds(kb * 16, 16)] = jnp.where(idxv >= slot, jnp.int32(S), old)
    scatter_flush()


@functools.partial(
    pl.kernel,
    out_type=(jax.ShapeDtypeStruct((2 * NW, D), jnp.float32),
              jax.ShapeDtypeStruct((1, 2 * NW), jnp.float32)),
    mesh=_mesh,
    scratch_types=[
        pltpu.VMEM((2 * NW, D), jnp.float32),
        pltpu.VMEM((2 * NW, 16), jnp.float32),
        pltpu.VMEM((2 * NW, D), jnp.float32),
        pltpu.VMEM((1, 2 * NW), jnp.float32),
    ],
)
def _merge_k(side_hbm, meta_hbm, bmean_hbm, bid_hbm, sbuf, mbuf, obuf, oidbuf):
    pltpu.sync_copy(side_hbm, sbuf)
    pltpu.sync_copy(meta_hbm, mbuf)
    lane_iota = lax.iota(jnp.int32, 16)

    def zero_row(e, _):
        for j in range(8):
            obuf[e, pl.ds(16 * j, 16)] = jnp.zeros((16,), jnp.float32)
        return 0

    lax.fori_loop(0, 2 * NW, zero_row, 0)

    def store_state(slot, idf, cnt, acc, pend):
        pend = jnp.where(lane_iota == jnp.bitwise_and(slot, 15), idf, pend)
        inv = (jnp.ones((16,), jnp.float32)
               / jnp.full((16,), jnp.maximum(cnt, jnp.float32(1.0)),
                          jnp.float32))
        for j in range(8):
            obuf[slot, pl.ds(16 * j, 16)] = acc[j] * inv
        blk16 = pl.multiple_of(jnp.right_shift(slot, 4) * 16, 16)
        oidbuf[0, pl.ds(blk16, 16)] = pend
        return pend

    def ent_step(e, st):
        (cur_id, cnt, own, slot, pend) = st[:5]
        acc = st[5:]
        mrow = mbuf[e, pl.ds(0, 16)]
        ecnt, eid, evalid = mrow[0], mrow[1], mrow[2]
        valid = evalid > jnp.float32(0.5)
        xs = tuple(sbuf[e, pl.ds(16 * j, 16)] for j in range(8))
        same = jnp.logical_or(eid == cur_id, jnp.logical_not(valid))
        do_flush = jnp.logical_and(jnp.logical_not(same), own)
        slot = slot + jnp.where(do_flush, 1, 0)
        gate = jnp.logical_and(valid, jnp.bool_(True))
        add = jnp.where(gate, jnp.float32(1.0), jnp.float32(0.0))
        nacc = tuple(jnp.where(same, a + x * add, x)
                     for a, x in zip(acc, xs))
        ncnt = jnp.where(same, cnt + ecnt * add, ecnt)
        nid = jnp.where(valid, eid, cur_id)
        nown = jnp.logical_or(own, jnp.logical_not(same))
        pend = store_state(slot, nid, ncnt, nacc, pend)
        return (nid, ncnt, nown, slot, pend) + nacc

    zero8 = tuple(jnp.zeros((16,), jnp.float32) for _ in range(8))
    st0 = ((jnp.float32(-1.0), jnp.float32(0.0), jnp.bool_(False),
            jnp.int32(0), jnp.zeros((16,), jnp.float32)) + zero8)
    st = lax.fori_loop(0, 2 * NW, ent_step, st0)
    own, slot, pend = st[2], st[3], st[4]
    slot_eff = slot + jnp.where(own, 1, 0)

    for kb in range((2 * NW) // 16):
        idxv = lane_iota + kb * 16
        old = oidbuf[0, pl.ds(kb * 16, 16)]
        oidbuf[0, pl.ds(kb * 16, 16)] = jnp.where(idxv >= slot_eff,
                                                  jnp.float32(-1.0), old)
    pltpu.sync_copy(obuf, bmean_hbm)
    pltpu.sync_copy(oidbuf, bid_hbm)


def _proj_body(x_ref, w1_ref, w2_ref, b1_ref, b2_ref, o_ref):
    hi = jax.lax.Precision.HIGHEST
    w = jnp.dot(w1_ref[...], w2_ref[...], precision=hi,
                preferred_element_type=jnp.float32)
    b = jnp.dot(b1_ref[...], w2_ref[...], precision=hi,
                preferred_element_type=jnp.float32) + b2_ref[...]
    o_ref[...] = jnp.dot(x_ref[...], w, precision=hi,
                         preferred_element_type=jnp.float32) + b


def _proj(x, w1, w2, b1, b2, blk):
    rows = x.shape[0]
    grid = rows // blk
    return pl.pallas_call(
        _proj_body,
        grid=(grid,),
        in_specs=[
            pl.BlockSpec((blk, D), lambda i: (i, 0)),
            pl.BlockSpec((D, D), lambda i: (0, 0)),
            pl.BlockSpec((D, D), lambda i: (0, 0)),
            pl.BlockSpec((1, D), lambda i: (0, 0)),
            pl.BlockSpec((1, D), lambda i: (0, 0)),
        ],
        out_specs=pl.BlockSpec((blk, D), lambda i: (i, 0)),
        out_shape=jax.ShapeDtypeStruct((rows, D), jnp.float32),
    )(x, w1, w2, b1.reshape(1, D), b2.reshape(1, D))


def _proj_patch_body(x_ref, bm_ref, bi_ref, w1_ref, w2_ref, b1_ref, b2_ref,
                     o_ref):
    hi = jax.lax.Precision.HIGHEST
    i = pl.program_id(0)
    blk = x_ref.shape[0]
    w = jnp.dot(w1_ref[...], w2_ref[...], precision=hi,
                preferred_element_type=jnp.float32)
    b = jnp.dot(b1_ref[...], w2_ref[...], precision=hi,
                preferred_element_type=jnp.float32) + b2_ref[...]
    rowid = (jax.lax.broadcasted_iota(jnp.int32, (blk, 2 * NW), 0)
             + blk * i).astype(jnp.float32)
    eq = (rowid == bi_ref[...]).astype(jnp.float32)
    corr = jnp.dot(eq, bm_ref[...], precision=hi,
                   preferred_element_type=jnp.float32)
    hasb = jnp.sum(eq, axis=1, keepdims=True)
    means = jnp.where(hasb > 0, corr, x_ref[...])
    o_ref[...] = jnp.dot(means, w, precision=hi,
                         preferred_element_type=jnp.float32) + b


def _proj_patch(x, bm, bi, w1, w2, b1, b2, blk):
    rows = x.shape[0]
    grid = rows // blk
    return pl.pallas_call(
        _proj_patch_body,
        grid=(grid,),
        in_specs=[
            pl.BlockSpec((blk, D), lambda i: (i, 0)),
            pl.BlockSpec((2 * NW, D), lambda i: (0, 0)),
            pl.BlockSpec((1, 2 * NW), lambda i: (0, 0)),
            pl.BlockSpec((D, D), lambda i: (0, 0)),
            pl.BlockSpec((D, D), lambda i: (0, 0)),
            pl.BlockSpec((1, D), lambda i: (0, 0)),
            pl.BlockSpec((1, D), lambda i: (0, 0)),
        ],
        out_specs=pl.BlockSpec((blk, D), lambda i: (i, 0)),
        out_shape=jax.ShapeDtypeStruct((rows, D), jnp.float32),
    )(x, bm, bi, w1, w2, b1.reshape(1, D), b2.reshape(1, D))


@functools.partial(
    pl.kernel,
    out_type=jax.ShapeDtypeStruct((N, D), jnp.float32),
    mesh=_mesh,
    scratch_types=[
        pltpu.VMEM((G,), jnp.int32),
        pltpu.VMEM((G, D), jnp.float32),
        pltpu.VMEM((G, D), jnp.float32),
        pltpu.SemaphoreType.DMA,
        pltpu.SemaphoreType.DMA,
    ],
)
def _gather_add_k(part_hbm, pd_hbm, ids_hbm, out_hbm, idbuf, pbuf, gbuf, sem,
                  sem2):
    wid = _wid()
    base = wid * NPW

    def do_chunk(pos, g):
        pos = pl.multiple_of(pos, 8)
        cp_p = pltpu.async_copy(part_hbm.at[pl.ds(pos, g), :],
                                pbuf.at[pl.ds(0, g), :], sem2)
        pltpu.sync_copy(ids_hbm.at[pl.ds(pos, g)], idbuf.at[pl.ds(0, g)])
        cp_g = pltpu.async_copy(pd_hbm.at[idbuf.at[pl.ds(0, g)]],
                                gbuf.at[pl.ds(0, g), :], sem)
        cp_p.wait()
        cp_g.wait()

        def row_body(r, _):
            for j in range(8):
                sl = pl.ds(16 * j, 16)
                pbuf[r, sl] = pbuf[r, sl] + gbuf[r, sl]
            return 0

        lax.fori_loop(0, g, row_body, 0)
        pltpu.sync_copy(pbuf.at[pl.ds(0, g), :], out_hbm.at[pl.ds(pos, g), :])

    def chunk_body(ch, _):
        do_chunk(base + ch * G, G)
        return 0

    lax.fori_loop(0, NG, chunk_body, 0)
    do_chunk(base + NG * G, GT)


def kernel(input_feat, segment_ids, W_in, b_in, W_down, b_down, W_skip, b_skip):
    ids32 = segment_ids.astype(jnp.int32)
    means, sides, metas = _seg_scan_k(input_feat, ids32)
    bmeans, bids = _merge_k(sides, metas)
    partial = _proj(input_feat, W_in, W_skip, b_in, b_skip, 2000)
    pd = _proj_patch(means[:S], bmeans, bids,
                     W_in, W_down, b_in, b_down, 2000)
    return _gather_add_k(partial, pd, ids32)
